# baseline (speedup 1.0000x reference)
import functools

import jax
import jax.numpy as jnp
from jax import lax
from jax.experimental import pallas as pl
from jax.experimental.pallas import tpu as pltpu
from jax.experimental.pallas import tpu_sc as plsc

VOCAB = 100000
EMBED_DIM = 16
BATCH = 16384
OUT_DIM = EMBED_DIM + 1

_INFO = plsc.get_sparse_core_info()
NUM_CORES = _INFO.num_cores
NUM_SUBCORES = _INFO.num_subcores
LANES = _INFO.num_lanes
NW = NUM_CORES * NUM_SUBCORES
B_PER_W = BATCH // NW
GATHER_CHUNK = 128
N_CHUNKS = B_PER_W // GATHER_CHUNK
OUT_WORDS = B_PER_W * OUT_DIM


def _tec_body(pitch_hbm, idx_hbm, table_hbm, out_hbm,
              idx_v, pitch_v, rows_v, out_v, idx_sem, sem):
  wid = lax.axis_index("s") * NUM_CORES + lax.axis_index("c")

  idx_copy = pltpu.make_async_copy(idx_hbm.at[wid], idx_v, idx_sem)
  idx_copy.start()
  pitch_copy = pltpu.make_async_copy(pitch_hbm.at[wid], pitch_v, idx_sem)
  pitch_copy.start()
  idx_copy.wait()

  copies = []
  for j in range(N_CHUNKS):
    copies.append(
        pltpu.make_async_copy(
            table_hbm.at[idx_v.at[j]],
            rows_v.at[pl.ds(j * GATHER_CHUNK, GATHER_CHUNK)],
            sem,
        )
    )
  for c in copies:
    c.start()
  pitch_copy.wait()

  iota = lax.iota(jnp.int32, LANES)

  def pitch_body(j, row_idx):
    p = pitch_v[pl.ds(j * LANES, LANES)]
    plsc.store_scatter(out_v, [row_idx * OUT_DIM], p)
    return row_idx + LANES

  lax.fori_loop(0, B_PER_W // LANES, pitch_body, iota, unroll=4)

  for c in copies:
    c.wait()

  def row_body(i, col_idx):
    v = rows_v[i]
    plsc.store_scatter(out_v, [col_idx], v)
    return col_idx + OUT_DIM

  lax.fori_loop(0, B_PER_W, row_body, iota + 1, unroll=8)

  pltpu.sync_copy(out_v, out_hbm.at[wid])


@jax.jit
def kernel(pitch, timbre_id, table):
  mesh = plsc.VectorSubcoreMesh(core_axis_name="c", subcore_axis_name="s")
  run = functools.partial(
      pl.kernel,
      mesh=mesh,
      out_type=jax.ShapeDtypeStruct((NW, OUT_WORDS), jnp.float32),
      scratch_types=[
          pltpu.VMEM((N_CHUNKS, GATHER_CHUNK), jnp.int32),
          pltpu.VMEM((B_PER_W,), jnp.float32),
          pltpu.VMEM((B_PER_W, EMBED_DIM), jnp.float32),
          pltpu.VMEM((OUT_WORDS,), jnp.float32),
          pltpu.SemaphoreType.DMA,
          pltpu.SemaphoreType.DMA,
      ],
      compiler_params=pltpu.CompilerParams(
          needs_layout_passes=False, use_tc_tiling_on_sc=False),
  )(_tec_body)
  out = run(
      pitch.reshape(NW, B_PER_W),
      timbre_id.reshape(NW, N_CHUNKS, GATHER_CHUNK),
      table,
  )
  return out.reshape(BATCH, OUT_DIM)


# speedup vs baseline: 1.0393x; 1.0393x over previous
import functools

import jax
import jax.numpy as jnp
from jax import lax
from jax.experimental import pallas as pl
from jax.experimental.pallas import tpu as pltpu
from jax.experimental.pallas import tpu_sc as plsc

VOCAB = 100000
EMBED_DIM = 16
BATCH = 16384
OUT_DIM = EMBED_DIM + 1

_INFO = plsc.get_sparse_core_info()
NUM_CORES = _INFO.num_cores
NUM_SUBCORES = _INFO.num_subcores
LANES = _INFO.num_lanes
NW = NUM_CORES * NUM_SUBCORES
B_PER_W = BATCH // NW
GATHER_CHUNK = 128
N_CHUNKS = B_PER_W // GATHER_CHUNK


def _tec_body(pitch_hbm, idx_hbm, table_hbm, out_hbm,
              idx_v, pitch_v, rows_v, out_v, idx_sem, sem):
  wid = lax.axis_index("s") * NUM_CORES + lax.axis_index("c")

  idx_copy = pltpu.make_async_copy(idx_hbm.at[wid], idx_v, idx_sem)
  idx_copy.start()
  pitch_copy = pltpu.make_async_copy(pitch_hbm.at[wid], pitch_v, idx_sem)
  pitch_copy.start()
  idx_copy.wait()

  copies = []
  for j in range(N_CHUNKS):
    copies.append(
        pltpu.make_async_copy(
            table_hbm.at[idx_v.at[j]],
            rows_v.at[pl.ds(j * GATHER_CHUNK, GATHER_CHUNK)],
            sem,
        )
    )
  for c in copies:
    c.start()
  pitch_copy.wait()

  iota = lax.iota(jnp.int32, LANES)
  zeros = iota - iota

  def pitch_body(j, row_idx):
    p = pitch_v[pl.ds(j * LANES, LANES)]
    plsc.store_scatter(out_v, [row_idx, zeros], p)
    return row_idx + LANES

  lax.fori_loop(0, B_PER_W // LANES, pitch_body, iota, unroll=4)

  for c in copies:
    c.wait()

  def row_body(i, row_idx):
    v = rows_v[i]
    plsc.store_scatter(out_v, [row_idx, iota + 1], v)
    return row_idx + 1

  lax.fori_loop(0, B_PER_W, row_body, zeros, unroll=8)

  pltpu.sync_copy(out_v, out_hbm.at[pl.ds(wid * B_PER_W, B_PER_W), :])


def _run_kernel(pitch, timbre_id, table):
  mesh = plsc.VectorSubcoreMesh(core_axis_name="c", subcore_axis_name="s")
  run = functools.partial(
      pl.kernel,
      mesh=mesh,
      out_type=jax.ShapeDtypeStruct((BATCH, OUT_DIM), jnp.float32),
      scratch_types=[
          pltpu.VMEM((N_CHUNKS, GATHER_CHUNK), jnp.int32),
          pltpu.VMEM((B_PER_W,), jnp.float32),
          pltpu.VMEM((B_PER_W, EMBED_DIM), jnp.float32),
          pltpu.VMEM((B_PER_W, OUT_DIM), jnp.float32),
          pltpu.SemaphoreType.DMA,
          pltpu.SemaphoreType.DMA,
      ],
      compiler_params=pltpu.CompilerParams(
          needs_layout_passes=False, use_tc_tiling_on_sc=False),
  )(_tec_body)
  return run(
      pitch.reshape(NW, B_PER_W),
      timbre_id.reshape(NW, N_CHUNKS, GATHER_CHUNK),
      table,
  )


kernel = jax.jit(_run_kernel)


# speedup vs baseline: 1.1609x; 1.1170x over previous
import functools

import jax
import jax.numpy as jnp
from jax import lax
from jax.experimental import pallas as pl
from jax.experimental.pallas import tpu as pltpu
from jax.experimental.pallas import tpu_sc as plsc

VOCAB = 100000
EMBED_DIM = 16
BATCH = 16384
OUT_DIM = EMBED_DIM + 1

_INFO = plsc.get_sparse_core_info()
NUM_CORES = _INFO.num_cores
NUM_SUBCORES = _INFO.num_subcores
LANES = _INFO.num_lanes
NW = NUM_CORES * NUM_SUBCORES
B_PER_W = BATCH // NW
GATHER_CHUNK = 128
N_CHUNKS = B_PER_W // GATHER_CHUNK


def _tec_body(pitch_hbm, idx_hbm, table_hbm, out_hbm,
              idx_v, rows_v, out_v, idx_sem, sem):
  wid = lax.axis_index("s") * NUM_CORES + lax.axis_index("c")
  base = wid * B_PER_W

  idx_copy = pltpu.make_async_copy(idx_hbm.at[wid], idx_v, idx_sem)
  idx_copy.start()
  pitch_copy = pltpu.make_async_copy(
      pitch_hbm.at[wid], out_v.at[pl.ds(0, 1), :], idx_sem)
  pitch_copy.start()
  idx_copy.wait()

  copies = []
  for j in range(N_CHUNKS):
    copies.append(
        pltpu.make_async_copy(
            table_hbm.at[idx_v.at[j]],
            rows_v.at[pl.ds(j * GATHER_CHUNK, GATHER_CHUNK)],
            sem,
        )
    )
  for c in copies:
    c.start()
  for c in copies:
    c.wait()

  iota = lax.iota(jnp.int32, LANES)
  zeros = iota - iota

  def row_body(i, col_idx):
    v = rows_v[i]
    plsc.store_scatter(out_v, [iota + 1, col_idx], v)
    return col_idx + 1

  lax.fori_loop(0, B_PER_W, row_body, zeros, unroll=8)

  pitch_copy.wait()

  pltpu.sync_copy(out_v, out_hbm.at[:, pl.ds(base, B_PER_W)])


def _run_kernel(pitch, timbre_id, table):
  mesh = plsc.VectorSubcoreMesh(core_axis_name="c", subcore_axis_name="s")
  run = functools.partial(
      pl.kernel,
      mesh=mesh,
      out_type=jax.ShapeDtypeStruct((OUT_DIM, BATCH), jnp.float32),
      scratch_types=[
          pltpu.VMEM((N_CHUNKS, GATHER_CHUNK), jnp.int32),
          pltpu.VMEM((B_PER_W, EMBED_DIM), jnp.float32),
          pltpu.VMEM((OUT_DIM, B_PER_W), jnp.float32),
          pltpu.SemaphoreType.DMA,
          pltpu.SemaphoreType.DMA,
      ],
      compiler_params=pltpu.CompilerParams(
          needs_layout_passes=False, use_tc_tiling_on_sc=False),
  )(_tec_body)
  out = run(
      pitch.reshape(NW, 1, B_PER_W),
      timbre_id.reshape(NW, N_CHUNKS, GATHER_CHUNK),
      table,
  )
  return out.T


kernel = jax.jit(_run_kernel)


# speedup vs baseline: 1.1653x; 1.0038x over previous
import functools

import jax
import jax.numpy as jnp
from jax import lax
from jax.experimental import pallas as pl
from jax.experimental.pallas import tpu as pltpu
from jax.experimental.pallas import tpu_sc as plsc

VOCAB = 100000
EMBED_DIM = 16
BATCH = 16384
OUT_DIM = EMBED_DIM + 1

_INFO = plsc.get_sparse_core_info()
NUM_CORES = _INFO.num_cores
NUM_SUBCORES = _INFO.num_subcores
LANES = _INFO.num_lanes
NW = NUM_CORES * NUM_SUBCORES
B_PER_W = BATCH // NW
GATHER_CHUNK = 128
N_CHUNKS = B_PER_W // GATHER_CHUNK


def _tec_body(pitch_hbm, idx_hbm, table_hbm, out_hbm,
              idx_v, rows_v, out_v, idx_sem, sem):
  wid = lax.axis_index("s") * NUM_CORES + lax.axis_index("c")
  base = wid * B_PER_W

  idx_copy = pltpu.make_async_copy(idx_hbm.at[wid], idx_v, idx_sem)
  idx_copy.start()
  pitch_copy = pltpu.make_async_copy(
      pitch_hbm.at[wid], out_v.at[pl.ds(0, 1), :], idx_sem)
  pitch_copy.start()
  idx_copy.wait()

  copies = []
  for j in range(N_CHUNKS):
    copies.append(
        pltpu.make_async_copy(
            table_hbm.at[idx_v.at[j]],
            rows_v.at[pl.ds(j * GATHER_CHUNK, GATHER_CHUNK)],
            sem,
        )
    )
  for c in copies:
    c.start()
  for c in copies:
    c.wait()

  iota = lax.iota(jnp.int32, LANES)
  zeros = iota - iota

  def row_body(i, col_idx):
    v = rows_v[i]
    plsc.store_scatter(out_v, [iota + 1, col_idx], v)
    return col_idx + 1

  lax.fori_loop(0, B_PER_W, row_body, zeros, unroll=8)

  pitch_copy.wait()

  pltpu.sync_copy(out_v, out_hbm.at[:, pl.ds(base, B_PER_W)])


def _run_kernel(pitch, timbre_id, table):
  mesh = plsc.VectorSubcoreMesh(core_axis_name="c", subcore_axis_name="s")
  run = functools.partial(
      pl.kernel,
      mesh=mesh,
      out_type=jax.ShapeDtypeStruct((OUT_DIM, BATCH), jnp.float32),
      scratch_types=[
          pltpu.VMEM((N_CHUNKS, GATHER_CHUNK), jnp.int32),
          pltpu.VMEM((B_PER_W, EMBED_DIM), jnp.float32),
          pltpu.VMEM((OUT_DIM, B_PER_W), jnp.float32),
          pltpu.SemaphoreType.DMA,
          pltpu.SemaphoreType.DMA,
      ],
      compiler_params=pltpu.CompilerParams(
          needs_layout_passes=False, use_tc_tiling_on_sc=False,
          disable_bounds_checks=True, disable_semaphore_checks=True),
  )(_tec_body)
  out = run(
      pitch.reshape(NW, 1, B_PER_W),
      timbre_id.reshape(NW, N_CHUNKS, GATHER_CHUNK),
      table,
  )
  return out.T


kernel = jax.jit(_run_kernel)
